# baseline (speedup 1.0000x reference)
import functools

import jax
import jax.numpy as jnp
from jax import lax
from jax.experimental import pallas as pl
from jax.experimental.pallas import tpu as pltpu
from jax.experimental.pallas import tpu_sc as plsc

_LANES = 16


def _make_sc_kernel(B, NV, num_cores, num_subcores):
    NW = num_cores * num_subcores
    per_w = B // NW
    mesh = plsc.VectorSubcoreMesh(core_axis_name="c", subcore_axis_name="s")

    @functools.partial(
        pl.kernel,
        out_type=jax.ShapeDtypeStruct((B,), jnp.float32),
        mesh=mesh,
        scratch_types=[
            pltpu.VMEM((per_w,), jnp.int32),
            pltpu.VMEM((per_w,), jnp.int32),
            pltpu.VMEM((per_w,), jnp.float32),
            pltpu.SemaphoreType.DMA,
        ],
    )
    def body(x_hbm, y_hbm, tab_hbm, out_hbm, x_v, idx_v, val_v, sem):
        wid = lax.axis_index("s") * num_cores + lax.axis_index("c")
        base = wid * per_w

        pltpu.sync_copy(x_hbm.at[pl.ds(base, per_w)], x_v)
        pltpu.sync_copy(y_hbm.at[pl.ds(base, per_w)], idx_v)

        def mk_idx(i, carry):
            s = pl.ds(i * _LANES, _LANES)
            idx_v[s] = x_v[s] * NV + idx_v[s]
            return carry

        lax.fori_loop(0, per_w // _LANES, mk_idx, 0)

        pltpu.async_copy(tab_hbm.at[idx_v], val_v, sem).wait()

        def sig(i, carry):
            s = pl.ds(i * _LANES, _LANES)
            v = val_v[s]
            val_v[s] = 1.0 / (1.0 + jnp.exp(-v))
            return carry

        lax.fori_loop(0, per_w // _LANES, sig, 0)

        pltpu.sync_copy(val_v, out_hbm.at[pl.ds(base, per_w)])

    return body


def kernel(x, y, logits):
    N, M = logits.shape
    B = x.shape[0]
    info = plsc.get_sparse_core_info()
    flat = logits.reshape(N * M)
    run = _make_sc_kernel(B, M, info.num_cores, info.num_subcores)
    return run(x.astype(jnp.int32), y.astype(jnp.int32), flat)
